# Initial kernel scaffold; baseline (speedup 1.0000x reference)
import jax
import jax.numpy as jnp
from jax.experimental import pallas as pl


def kernel(idx, emb1, emb2, W1, b1, W2, b2):
    raise NotImplementedError("write your pallas kernel here")


# speedup vs baseline: 23.2510x; 23.2510x over previous
import functools

import jax
import jax.numpy as jnp
from jax import lax
from jax.experimental import pallas as pl

ALPHA = 3.0
K = 64
STRIP = 256


def _dense_tanh_kernel(e1_ref, w1_ref, b1_ref, e2_ref, w2_ref, b2_ref,
                       v1_ref, v2_ref):
    e1 = e1_ref[0]
    w1 = w1_ref[0]
    e2 = e2_ref[0]
    w2 = w2_ref[0]
    h1 = lax.dot_general(e1, w1, (((1,), (1,)), ((), ())),
                         preferred_element_type=jnp.float32) + b1_ref[0, 0]
    h2 = lax.dot_general(e2, w2, (((1,), (1,)), ((), ())),
                         preferred_element_type=jnp.float32) + b2_ref[0, 0]
    v1_ref[0] = jnp.tanh(ALPHA * h1)
    v2_ref[0] = jnp.tanh(ALPHA * h2)


def _adj_block(v1s, v2full):
    t1 = lax.dot_general(v1s, v2full, (((1,), (1,)), ((), ())),
                         preferred_element_type=jnp.float32)
    t2 = lax.dot_general(v2full, v1s, (((1,), (1,)), ((), ())),
                         preferred_element_type=jnp.float32)
    a = t1 - t2.T
    return jax.nn.relu(jnp.tanh(ALPHA * a))


def _strip_kernel(v1_ref, v2_ref, o_ref):
    adj = jnp.concatenate(
        [_adj_block(v1_ref[0], v2_ref[0]), _adj_block(v1_ref[1], v2_ref[1])],
        axis=1)

    lo = jnp.full((STRIP, 1), -1.0, jnp.float32)
    hi = jnp.full((STRIP, 1), 1.0, jnp.float32)
    for _ in range(26):
        mid = 0.5 * (lo + hi)
        cnt = jnp.sum((adj > mid).astype(jnp.float32), axis=1, keepdims=True)
        ge_k = cnt >= K
        lo = jnp.where(ge_k, mid, lo)
        hi = jnp.where(ge_k, hi, mid)
    o_ref[...] = jnp.where(adj > lo, adj, 0.0)


def kernel(idx, emb1, emb2, W1, b1, W2, b2):
    nm, n_sub, dim = emb1.shape
    n_mod = 2
    N = n_mod * n_sub

    v1, v2 = pl.pallas_call(
        _dense_tanh_kernel,
        out_shape=(jax.ShapeDtypeStruct((nm, n_sub, dim), jnp.float32),
                   jax.ShapeDtypeStruct((nm, n_sub, dim), jnp.float32)),
        grid=(nm,),
        in_specs=[
            pl.BlockSpec((1, n_sub, dim), lambda b: (b, 0, 0)),
            pl.BlockSpec((1, dim, dim), lambda b: (b, 0, 0)),
            pl.BlockSpec((1, 1, dim), lambda b: (b, 0, 0)),
            pl.BlockSpec((1, n_sub, dim), lambda b: (b, 0, 0)),
            pl.BlockSpec((1, dim, dim), lambda b: (b, 0, 0)),
            pl.BlockSpec((1, 1, dim), lambda b: (b, 0, 0)),
        ],
        out_specs=(pl.BlockSpec((1, n_sub, dim), lambda b: (b, 0, 0)),
                   pl.BlockSpec((1, n_sub, dim), lambda b: (b, 0, 0))),
    )(emb1, W1, b1.reshape(nm, 1, dim), emb2, W2, b2.reshape(nm, 1, dim))

    strips_per_mod = n_sub // STRIP
    out = pl.pallas_call(
        _strip_kernel,
        out_shape=jax.ShapeDtypeStruct((N, N), jnp.float32),
        grid=(n_mod * strips_per_mod,),
        in_specs=[
            pl.BlockSpec(
                (n_mod, STRIP, dim),
                lambda s: (s // strips_per_mod, s % strips_per_mod, 0)),
            pl.BlockSpec((n_mod, n_sub, dim),
                         lambda s: (s // strips_per_mod, 0, 0)),
        ],
        out_specs=pl.BlockSpec((STRIP, N), lambda s: (s, 0)),
    )(v1, v2)
    return out


# speedup vs baseline: 23.2652x; 1.0006x over previous
import functools

import jax
import jax.numpy as jnp
from jax import lax
from jax.experimental import pallas as pl
from jax.experimental.pallas import tpu as pltpu

ALPHA = 3.0
K = 64
STRIP = 256


def _dense_tanh_kernel(e1_ref, w1_ref, b1_ref, e2_ref, w2_ref, b2_ref,
                       v1_ref, v2_ref):
    e1 = e1_ref[0]
    w1 = w1_ref[0]
    e2 = e2_ref[0]
    w2 = w2_ref[0]
    h1 = lax.dot_general(e1, w1, (((1,), (1,)), ((), ())),
                         preferred_element_type=jnp.float32) + b1_ref[0, 0]
    h2 = lax.dot_general(e2, w2, (((1,), (1,)), ((), ())),
                         preferred_element_type=jnp.float32) + b2_ref[0, 0]
    v1_ref[0] = jnp.tanh(ALPHA * h1)
    v2_ref[0] = jnp.tanh(ALPHA * h2)


def _adj_block(v1s, v2full):
    t1 = lax.dot_general(v1s, v2full, (((1,), (1,)), ((), ())),
                         preferred_element_type=jnp.float32)
    t2 = lax.dot_general(v2full, v1s, (((1,), (1,)), ((), ())),
                         preferred_element_type=jnp.float32)
    a = t1 - t2.T
    return jax.nn.relu(jnp.tanh(ALPHA * a))


def _strip_kernel(v1_ref, v2_ref, o_ref):
    adj = jnp.concatenate(
        [_adj_block(v1_ref[0], v2_ref[0]), _adj_block(v1_ref[1], v2_ref[1])],
        axis=1)

    lo = jnp.full((STRIP, 1), -1.0, jnp.float32)
    hi = jnp.full((STRIP, 1), 1.0, jnp.float32)
    for _ in range(26):
        mid = 0.5 * (lo + hi)
        cnt = jnp.sum((adj > mid).astype(jnp.float32), axis=1, keepdims=True)
        ge_k = cnt >= K
        lo = jnp.where(ge_k, mid, lo)
        hi = jnp.where(ge_k, hi, mid)
    o_ref[...] = jnp.where(adj > lo, adj, 0.0)


def kernel(idx, emb1, emb2, W1, b1, W2, b2):
    nm, n_sub, dim = emb1.shape
    n_mod = 2
    N = n_mod * n_sub

    v1, v2 = pl.pallas_call(
        _dense_tanh_kernel,
        out_shape=(jax.ShapeDtypeStruct((nm, n_sub, dim), jnp.float32),
                   jax.ShapeDtypeStruct((nm, n_sub, dim), jnp.float32)),
        grid=(nm,),
        in_specs=[
            pl.BlockSpec((1, n_sub, dim), lambda b: (b, 0, 0)),
            pl.BlockSpec((1, dim, dim), lambda b: (b, 0, 0)),
            pl.BlockSpec((1, 1, dim), lambda b: (b, 0, 0)),
            pl.BlockSpec((1, n_sub, dim), lambda b: (b, 0, 0)),
            pl.BlockSpec((1, dim, dim), lambda b: (b, 0, 0)),
            pl.BlockSpec((1, 1, dim), lambda b: (b, 0, 0)),
        ],
        out_specs=(pl.BlockSpec((1, n_sub, dim), lambda b: (b, 0, 0)),
                   pl.BlockSpec((1, n_sub, dim), lambda b: (b, 0, 0))),
        compiler_params=pltpu.CompilerParams(
            dimension_semantics=("parallel",)),
    )(emb1, W1, b1.reshape(nm, 1, dim), emb2, W2, b2.reshape(nm, 1, dim))

    strips_per_mod = n_sub // STRIP
    out = pl.pallas_call(
        _strip_kernel,
        out_shape=jax.ShapeDtypeStruct((N, N), jnp.float32),
        grid=(n_mod * strips_per_mod,),
        in_specs=[
            pl.BlockSpec(
                (n_mod, STRIP, dim),
                lambda s: (s // strips_per_mod, s % strips_per_mod, 0)),
            pl.BlockSpec((n_mod, n_sub, dim),
                         lambda s: (s // strips_per_mod, 0, 0)),
        ],
        out_specs=pl.BlockSpec((STRIP, N), lambda s: (s, 0)),
        compiler_params=pltpu.CompilerParams(
            dimension_semantics=("parallel",)),
    )(v1, v2)
    return out


# speedup vs baseline: 33.0552x; 1.4208x over previous
import functools

import jax
import jax.numpy as jnp
from jax import lax
from jax.experimental import pallas as pl
from jax.experimental.pallas import tpu as pltpu

ALPHA = 3.0
K = 64
STRIP = 256


def _dense_tanh_kernel(e1_ref, w1_ref, b1_ref, e2_ref, w2_ref, b2_ref,
                       v1_ref, v2_ref):
    e1 = e1_ref[0]
    w1 = w1_ref[0]
    e2 = e2_ref[0]
    w2 = w2_ref[0]
    h1 = lax.dot_general(e1, w1, (((1,), (1,)), ((), ())),
                         preferred_element_type=jnp.float32) + b1_ref[0, 0]
    h2 = lax.dot_general(e2, w2, (((1,), (1,)), ((), ())),
                         preferred_element_type=jnp.float32) + b2_ref[0, 0]
    v1_ref[0] = jnp.tanh(ALPHA * h1)
    v2_ref[0] = jnp.tanh(ALPHA * h2)


def _adj_block(v1s, v2full):
    t1 = lax.dot_general(v1s, v2full, (((1,), (1,)), ((), ())),
                         preferred_element_type=jnp.float32)
    t2 = lax.dot_general(v2full, v1s, (((1,), (1,)), ((), ())),
                         preferred_element_type=jnp.float32)
    a = t1 - t2.T
    return jax.nn.relu(jnp.tanh(ALPHA * a))


def _strip_kernel(v1_ref, v2_ref, o_ref):
    adj = jnp.concatenate(
        [_adj_block(v1_ref[0], v2_ref[0]), _adj_block(v1_ref[1], v2_ref[1])],
        axis=1)

    adj16 = adj.astype(jnp.bfloat16)
    one16 = jnp.bfloat16(1.0)
    zero16 = jnp.bfloat16(0.0)
    lo = jnp.full((STRIP, 1), -1.0, jnp.float32)
    hi = jnp.full((STRIP, 1), 1.0, jnp.float32)
    for _ in range(16):
        mid = 0.5 * (lo + hi)
        cnt = jnp.sum(jnp.where(adj16 > mid.astype(jnp.bfloat16),
                                one16, zero16), axis=1, keepdims=True)
        ge_k = cnt.astype(jnp.float32) >= K
        lo = jnp.where(ge_k, mid, lo)
        hi = jnp.where(ge_k, hi, mid)
    o_ref[...] = jnp.where(adj > lo, adj, 0.0)


def kernel(idx, emb1, emb2, W1, b1, W2, b2):
    nm, n_sub, dim = emb1.shape
    n_mod = 2
    N = n_mod * n_sub

    v1, v2 = pl.pallas_call(
        _dense_tanh_kernel,
        out_shape=(jax.ShapeDtypeStruct((nm, n_sub, dim), jnp.float32),
                   jax.ShapeDtypeStruct((nm, n_sub, dim), jnp.float32)),
        grid=(nm,),
        in_specs=[
            pl.BlockSpec((1, n_sub, dim), lambda b: (b, 0, 0)),
            pl.BlockSpec((1, dim, dim), lambda b: (b, 0, 0)),
            pl.BlockSpec((1, 1, dim), lambda b: (b, 0, 0)),
            pl.BlockSpec((1, n_sub, dim), lambda b: (b, 0, 0)),
            pl.BlockSpec((1, dim, dim), lambda b: (b, 0, 0)),
            pl.BlockSpec((1, 1, dim), lambda b: (b, 0, 0)),
        ],
        out_specs=(pl.BlockSpec((1, n_sub, dim), lambda b: (b, 0, 0)),
                   pl.BlockSpec((1, n_sub, dim), lambda b: (b, 0, 0))),
        compiler_params=pltpu.CompilerParams(
            dimension_semantics=("parallel",)),
    )(emb1, W1, b1.reshape(nm, 1, dim), emb2, W2, b2.reshape(nm, 1, dim))

    strips_per_mod = n_sub // STRIP
    out = pl.pallas_call(
        _strip_kernel,
        out_shape=jax.ShapeDtypeStruct((N, N), jnp.float32),
        grid=(n_mod * strips_per_mod,),
        in_specs=[
            pl.BlockSpec(
                (n_mod, STRIP, dim),
                lambda s: (s // strips_per_mod, s % strips_per_mod, 0)),
            pl.BlockSpec((n_mod, n_sub, dim),
                         lambda s: (s // strips_per_mod, 0, 0)),
        ],
        out_specs=pl.BlockSpec((STRIP, N), lambda s: (s, 0)),
        compiler_params=pltpu.CompilerParams(
            dimension_semantics=("parallel",)),
    )(v1, v2)
    return out


# speedup vs baseline: 55.2508x; 1.6715x over previous
import functools

import jax
import jax.numpy as jnp
from jax import lax
from jax.experimental import pallas as pl
from jax.experimental.pallas import tpu as pltpu

ALPHA = 3.0
K = 64
STRIP = 256


def _dense_tanh_kernel(e1_ref, w1_ref, b1_ref, e2_ref, w2_ref, b2_ref,
                       v1_ref, v2_ref):
    e1 = e1_ref[0]
    w1 = w1_ref[0]
    e2 = e2_ref[0]
    w2 = w2_ref[0]
    h1 = lax.dot_general(e1, w1, (((1,), (1,)), ((), ())),
                         preferred_element_type=jnp.float32) + b1_ref[0, 0]
    h2 = lax.dot_general(e2, w2, (((1,), (1,)), ((), ())),
                         preferred_element_type=jnp.float32) + b2_ref[0, 0]
    v1_ref[0] = jnp.tanh(ALPHA * h1)
    v2_ref[0] = jnp.tanh(ALPHA * h2)


def _adj_block(v1s, v2full):
    t1 = lax.dot_general(v1s, v2full, (((1,), (1,)), ((), ())),
                         preferred_element_type=jnp.float32)
    t2 = lax.dot_general(v2full, v1s, (((1,), (1,)), ((), ())),
                         preferred_element_type=jnp.float32)
    a = t1 - t2.T
    return jax.nn.relu(jnp.tanh(ALPHA * a))


def _strip_kernel(v1_ref, v2_ref, o_ref):
    adj = jnp.concatenate(
        [_adj_block(v1_ref[0], v2_ref[0]), _adj_block(v1_ref[1], v2_ref[1])],
        axis=1)

    adj16 = adj.astype(jnp.bfloat16)
    one16 = jnp.bfloat16(1.0)
    zero16 = jnp.bfloat16(0.0)
    lo = jnp.full((STRIP, 1), -1.0, jnp.float32)
    hi = jnp.full((STRIP, 1), 1.0, jnp.float32)
    for _ in range(12):
        mid = 0.5 * (lo + hi)
        c = jnp.where(adj16 > mid.astype(jnp.bfloat16), one16, zero16)
        w = c.shape[1]
        while w > 128:
            w //= 2
            c = c[:, :w] + c[:, w:2 * w]
        cnt = jnp.sum(c.astype(jnp.float32), axis=1, keepdims=True)
        ge_k = cnt >= K
        lo = jnp.where(ge_k, mid, lo)
        hi = jnp.where(ge_k, hi, mid)
    o_ref[...] = jnp.where(adj > lo, adj, 0.0)


def kernel(idx, emb1, emb2, W1, b1, W2, b2):
    nm, n_sub, dim = emb1.shape
    n_mod = 2
    N = n_mod * n_sub

    v1, v2 = pl.pallas_call(
        _dense_tanh_kernel,
        out_shape=(jax.ShapeDtypeStruct((nm, n_sub, dim), jnp.float32),
                   jax.ShapeDtypeStruct((nm, n_sub, dim), jnp.float32)),
        grid=(nm,),
        in_specs=[
            pl.BlockSpec((1, n_sub, dim), lambda b: (b, 0, 0)),
            pl.BlockSpec((1, dim, dim), lambda b: (b, 0, 0)),
            pl.BlockSpec((1, 1, dim), lambda b: (b, 0, 0)),
            pl.BlockSpec((1, n_sub, dim), lambda b: (b, 0, 0)),
            pl.BlockSpec((1, dim, dim), lambda b: (b, 0, 0)),
            pl.BlockSpec((1, 1, dim), lambda b: (b, 0, 0)),
        ],
        out_specs=(pl.BlockSpec((1, n_sub, dim), lambda b: (b, 0, 0)),
                   pl.BlockSpec((1, n_sub, dim), lambda b: (b, 0, 0))),
        compiler_params=pltpu.CompilerParams(
            dimension_semantics=("parallel",)),
    )(emb1, W1, b1.reshape(nm, 1, dim), emb2, W2, b2.reshape(nm, 1, dim))

    strips_per_mod = n_sub // STRIP
    out = pl.pallas_call(
        _strip_kernel,
        out_shape=jax.ShapeDtypeStruct((N, N), jnp.float32),
        grid=(n_mod * strips_per_mod,),
        in_specs=[
            pl.BlockSpec(
                (n_mod, STRIP, dim),
                lambda s: (s // strips_per_mod, s % strips_per_mod, 0)),
            pl.BlockSpec((n_mod, n_sub, dim),
                         lambda s: (s // strips_per_mod, 0, 0)),
        ],
        out_specs=pl.BlockSpec((STRIP, N), lambda s: (s, 0)),
        compiler_params=pltpu.CompilerParams(
            dimension_semantics=("parallel",)),
    )(v1, v2)
    return out


# speedup vs baseline: 62.7063x; 1.1349x over previous
import functools

import jax
import jax.numpy as jnp
from jax import lax
from jax.experimental import pallas as pl
from jax.experimental.pallas import tpu as pltpu

ALPHA = 3.0
K = 64
STRIP = 512


def _dense_tanh_kernel(e1_ref, w1_ref, b1_ref, e2_ref, w2_ref, b2_ref,
                       v1_ref, v2_ref):
    e1 = e1_ref[0]
    w1 = w1_ref[0]
    e2 = e2_ref[0]
    w2 = w2_ref[0]
    h1 = lax.dot_general(e1, w1, (((1,), (1,)), ((), ())),
                         preferred_element_type=jnp.float32) + b1_ref[0, 0]
    h2 = lax.dot_general(e2, w2, (((1,), (1,)), ((), ())),
                         preferred_element_type=jnp.float32) + b2_ref[0, 0]
    v1_ref[0] = jnp.tanh(ALPHA * h1)
    v2_ref[0] = jnp.tanh(ALPHA * h2)


def _adj_block(v1s, v2full):
    t1 = lax.dot_general(v1s, v2full, (((1,), (1,)), ((), ())),
                         preferred_element_type=jnp.float32)
    t2 = lax.dot_general(v2full, v1s, (((1,), (1,)), ((), ())),
                         preferred_element_type=jnp.float32)
    a = t1 - t2.T
    return jax.nn.relu(jnp.tanh(ALPHA * a))


def _strip_kernel(v1_ref, v2_ref, o_ref):
    adj = jnp.concatenate(
        [_adj_block(v1_ref[0], v2_ref[0]), _adj_block(v1_ref[1], v2_ref[1])],
        axis=1)

    adj16 = adj.astype(jnp.bfloat16)
    one16 = jnp.bfloat16(1.0)
    zero16 = jnp.bfloat16(0.0)
    lo = jnp.full((STRIP, 1), -1.0, jnp.float32)
    hi = jnp.full((STRIP, 1), 1.0, jnp.float32)
    for _ in range(10):
        mid = 0.5 * (lo + hi)
        c = jnp.where(adj16 > mid.astype(jnp.bfloat16), one16, zero16)
        w = c.shape[1]
        while w > 128:
            w //= 2
            c = c[:, :w] + c[:, w:2 * w]
        cnt = jnp.sum(c.astype(jnp.float32), axis=1, keepdims=True)
        ge_k = cnt >= K
        lo = jnp.where(ge_k, mid, lo)
        hi = jnp.where(ge_k, hi, mid)
    o_ref[...] = jnp.where(adj > lo, adj, 0.0)


def kernel(idx, emb1, emb2, W1, b1, W2, b2):
    nm, n_sub, dim = emb1.shape
    n_mod = 2
    N = n_mod * n_sub

    v1, v2 = pl.pallas_call(
        _dense_tanh_kernel,
        out_shape=(jax.ShapeDtypeStruct((nm, n_sub, dim), jnp.float32),
                   jax.ShapeDtypeStruct((nm, n_sub, dim), jnp.float32)),
        grid=(nm,),
        in_specs=[
            pl.BlockSpec((1, n_sub, dim), lambda b: (b, 0, 0)),
            pl.BlockSpec((1, dim, dim), lambda b: (b, 0, 0)),
            pl.BlockSpec((1, 1, dim), lambda b: (b, 0, 0)),
            pl.BlockSpec((1, n_sub, dim), lambda b: (b, 0, 0)),
            pl.BlockSpec((1, dim, dim), lambda b: (b, 0, 0)),
            pl.BlockSpec((1, 1, dim), lambda b: (b, 0, 0)),
        ],
        out_specs=(pl.BlockSpec((1, n_sub, dim), lambda b: (b, 0, 0)),
                   pl.BlockSpec((1, n_sub, dim), lambda b: (b, 0, 0))),
        compiler_params=pltpu.CompilerParams(
            dimension_semantics=("parallel",)),
    )(emb1, W1, b1.reshape(nm, 1, dim), emb2, W2, b2.reshape(nm, 1, dim))

    strips_per_mod = n_sub // STRIP
    out = pl.pallas_call(
        _strip_kernel,
        out_shape=jax.ShapeDtypeStruct((N, N), jnp.float32),
        grid=(n_mod * strips_per_mod,),
        in_specs=[
            pl.BlockSpec(
                (n_mod, STRIP, dim),
                lambda s: (s // strips_per_mod, s % strips_per_mod, 0)),
            pl.BlockSpec((n_mod, n_sub, dim),
                         lambda s: (s // strips_per_mod, 0, 0)),
        ],
        out_specs=pl.BlockSpec((STRIP, N), lambda s: (s, 0)),
        compiler_params=pltpu.CompilerParams(
            dimension_semantics=("parallel",)),
    )(v1, v2)
    return out


# speedup vs baseline: 63.1240x; 1.0067x over previous
import jax
import jax.numpy as jnp
from jax import lax
from jax.experimental import pallas as pl
from jax.experimental.pallas import tpu as pltpu

ALPHA = 3.0
K = 64
STRIP = 512
N_ITERS = 10


def _dense_tanh_kernel(e1_ref, w1_ref, b1_ref, e2_ref, w2_ref, b2_ref,
                       v1_ref, v2_ref):
    e1 = e1_ref[0]
    w1 = w1_ref[0]
    e2 = e2_ref[0]
    w2 = w2_ref[0]
    h1 = lax.dot_general(e1, w1, (((1,), (1,)), ((), ())),
                         preferred_element_type=jnp.float32) + b1_ref[0, 0]
    h2 = lax.dot_general(e2, w2, (((1,), (1,)), ((), ())),
                         preferred_element_type=jnp.float32) + b2_ref[0, 0]
    v1_ref[0] = (ALPHA * jnp.tanh(ALPHA * h1)).astype(jnp.bfloat16)
    v2_ref[0] = jnp.tanh(ALPHA * h2).astype(jnp.bfloat16)


def _t_block(v1s, v2full):
    t1 = lax.dot_general(v1s, v2full, (((1,), (1,)), ((), ())),
                         preferred_element_type=jnp.float32)
    t2 = lax.dot_general(v2full, v1s, (((1,), (1,)), ((), ())),
                         preferred_element_type=jnp.float32)
    return jnp.tanh(t1 - t2.T)


def _count_gt(c16, mid16):
    c = jnp.where(c16 > mid16, jnp.bfloat16(1.0), jnp.bfloat16(0.0))
    w = c.shape[1]
    while w > 128:
        w //= 2
        c = c[:, :w] + c[:, w:2 * w]
    return jnp.sum(c.astype(jnp.float32), axis=1, keepdims=True)


def _strip_kernel(v1_ref, v2_ref, o_ref):
    t = jnp.concatenate(
        [_t_block(v1_ref[0], v2_ref[0]), _t_block(v1_ref[1], v2_ref[1])],
        axis=1)
    t16 = t.astype(jnp.bfloat16)

    lo = jnp.full((STRIP, 1), -1.0, jnp.float32)
    hi = jnp.full((STRIP, 1), 1.0, jnp.float32)
    for _ in range(N_ITERS):
        mid = 0.5 * (lo + hi)
        cnt = _count_gt(t16, mid.astype(jnp.bfloat16))
        ge_k = cnt >= K
        lo = jnp.where(ge_k, mid, lo)
        hi = jnp.where(ge_k, hi, mid)
    thr = jnp.maximum(lo, 0.0)
    o_ref[...] = jnp.where(t > thr, t, 0.0)


def kernel(idx, emb1, emb2, W1, b1, W2, b2):
    nm, n_sub, dim = emb1.shape
    n_mod = 2
    N = n_mod * n_sub

    v1, v2 = pl.pallas_call(
        _dense_tanh_kernel,
        out_shape=(jax.ShapeDtypeStruct((nm, n_sub, dim), jnp.bfloat16),
                   jax.ShapeDtypeStruct((nm, n_sub, dim), jnp.bfloat16)),
        grid=(nm,),
        in_specs=[
            pl.BlockSpec((1, n_sub, dim), lambda b: (b, 0, 0)),
            pl.BlockSpec((1, dim, dim), lambda b: (b, 0, 0)),
            pl.BlockSpec((1, 1, dim), lambda b: (b, 0, 0)),
            pl.BlockSpec((1, n_sub, dim), lambda b: (b, 0, 0)),
            pl.BlockSpec((1, dim, dim), lambda b: (b, 0, 0)),
            pl.BlockSpec((1, 1, dim), lambda b: (b, 0, 0)),
        ],
        out_specs=(pl.BlockSpec((1, n_sub, dim), lambda b: (b, 0, 0)),
                   pl.BlockSpec((1, n_sub, dim), lambda b: (b, 0, 0))),
        compiler_params=pltpu.CompilerParams(
            dimension_semantics=("parallel",)),
    )(emb1, W1, b1.reshape(nm, 1, dim), emb2, W2, b2.reshape(nm, 1, dim))

    strips_per_mod = n_sub // STRIP
    out = pl.pallas_call(
        _strip_kernel,
        out_shape=jax.ShapeDtypeStruct((N, N), jnp.float32),
        grid=(n_mod * strips_per_mod,),
        in_specs=[
            pl.BlockSpec(
                (n_mod, STRIP, dim),
                lambda s: (s // strips_per_mod, s % strips_per_mod, 0)),
            pl.BlockSpec((n_mod, n_sub, dim),
                         lambda s: (s // strips_per_mod, 0, 0)),
        ],
        out_specs=pl.BlockSpec((STRIP, N), lambda s: (s, 0)),
        compiler_params=pltpu.CompilerParams(
            dimension_semantics=("parallel",)),
    )(v1, v2)
    return out


# speedup vs baseline: 68.4520x; 1.0844x over previous
import jax
import jax.numpy as jnp
from jax import lax
from jax.experimental import pallas as pl
from jax.experimental.pallas import tpu as pltpu

ALPHA = 3.0
K = 64
STRIP = 512
N_ITERS = 10


def _dense_tanh_kernel(e1_ref, w1_ref, b1_ref, e2_ref, w2_ref, b2_ref,
                       v1_ref, v2_ref):
    e1 = e1_ref[0]
    w1 = w1_ref[0]
    e2 = e2_ref[0]
    w2 = w2_ref[0]
    h1 = lax.dot_general(e1, w1, (((1,), (1,)), ((), ())),
                         preferred_element_type=jnp.float32) + b1_ref[0, 0]
    h2 = lax.dot_general(e2, w2, (((1,), (1,)), ((), ())),
                         preferred_element_type=jnp.float32) + b2_ref[0, 0]
    v1_ref[0] = (ALPHA * jnp.tanh(ALPHA * h1)).astype(jnp.bfloat16)
    v2_ref[0] = jnp.tanh(ALPHA * h2).astype(jnp.bfloat16)


def _t_block(v1s, v2full):
    t1 = lax.dot_general(v1s, v2full, (((1,), (1,)), ((), ())),
                         preferred_element_type=jnp.float32)
    t2 = lax.dot_general(v2full, v1s, (((1,), (1,)), ((), ())),
                         preferred_element_type=jnp.float32)
    t1_16 = t1.astype(jnp.bfloat16)
    t2_16 = t2.astype(jnp.bfloat16)
    return jnp.tanh(t1_16 - t2_16.T)


def _count_gt(c16, mid16):
    c = jnp.where(c16 > mid16, jnp.bfloat16(1.0), jnp.bfloat16(0.0))
    w = c.shape[1]
    while w > 128:
        w //= 2
        c = c[:, :w] + c[:, w:2 * w]
    return jnp.sum(c.astype(jnp.float32), axis=1, keepdims=True)


def _strip_kernel(v1_ref, v2_ref, o_ref):
    t16 = jnp.concatenate(
        [_t_block(v1_ref[0], v2_ref[0]), _t_block(v1_ref[1], v2_ref[1])],
        axis=1)

    lo = jnp.full((STRIP, 1), -1.0, jnp.float32)
    hi = jnp.full((STRIP, 1), 1.0, jnp.float32)
    for _ in range(N_ITERS):
        mid = 0.5 * (lo + hi)
        cnt = _count_gt(t16, mid.astype(jnp.bfloat16))
        ge_k = cnt >= K
        lo = jnp.where(ge_k, mid, lo)
        hi = jnp.where(ge_k, hi, mid)
    thr16 = jnp.maximum(lo, 0.0).astype(jnp.bfloat16)
    o_ref[...] = jnp.where(t16 > thr16, t16,
                           jnp.bfloat16(0.0)).astype(jnp.float32)


def kernel(idx, emb1, emb2, W1, b1, W2, b2):
    nm, n_sub, dim = emb1.shape
    n_mod = 2
    N = n_mod * n_sub

    v1, v2 = pl.pallas_call(
        _dense_tanh_kernel,
        out_shape=(jax.ShapeDtypeStruct((nm, n_sub, dim), jnp.bfloat16),
                   jax.ShapeDtypeStruct((nm, n_sub, dim), jnp.bfloat16)),
        grid=(nm,),
        in_specs=[
            pl.BlockSpec((1, n_sub, dim), lambda b: (b, 0, 0)),
            pl.BlockSpec((1, dim, dim), lambda b: (b, 0, 0)),
            pl.BlockSpec((1, 1, dim), lambda b: (b, 0, 0)),
            pl.BlockSpec((1, n_sub, dim), lambda b: (b, 0, 0)),
            pl.BlockSpec((1, dim, dim), lambda b: (b, 0, 0)),
            pl.BlockSpec((1, 1, dim), lambda b: (b, 0, 0)),
        ],
        out_specs=(pl.BlockSpec((1, n_sub, dim), lambda b: (b, 0, 0)),
                   pl.BlockSpec((1, n_sub, dim), lambda b: (b, 0, 0))),
        compiler_params=pltpu.CompilerParams(
            dimension_semantics=("parallel",)),
    )(emb1, W1, b1.reshape(nm, 1, dim), emb2, W2, b2.reshape(nm, 1, dim))

    strips_per_mod = n_sub // STRIP
    out = pl.pallas_call(
        _strip_kernel,
        out_shape=jax.ShapeDtypeStruct((N, N), jnp.float32),
        grid=(n_mod * strips_per_mod,),
        in_specs=[
            pl.BlockSpec(
                (n_mod, STRIP, dim),
                lambda s: (s // strips_per_mod, s % strips_per_mod, 0)),
            pl.BlockSpec((n_mod, n_sub, dim),
                         lambda s: (s // strips_per_mod, 0, 0)),
        ],
        out_specs=pl.BlockSpec((STRIP, N), lambda s: (s, 0)),
        compiler_params=pltpu.CompilerParams(
            dimension_semantics=("parallel",)),
    )(v1, v2)
    return out


# speedup vs baseline: 76.1744x; 1.1128x over previous
import jax
import jax.numpy as jnp
from jax import lax
from jax.experimental import pallas as pl
from jax.experimental.pallas import tpu as pltpu

ALPHA = 3.0
K = 64
STRIP = 512
N_ITERS = 8


def _dense_tanh_kernel(e1_ref, w1_ref, b1_ref, e2_ref, w2_ref, b2_ref,
                       v1_ref, v2_ref):
    e1 = e1_ref[0]
    w1 = w1_ref[0]
    e2 = e2_ref[0]
    w2 = w2_ref[0]
    h1 = lax.dot_general(e1, w1, (((1,), (1,)), ((), ())),
                         preferred_element_type=jnp.float32) + b1_ref[0, 0]
    h2 = lax.dot_general(e2, w2, (((1,), (1,)), ((), ())),
                         preferred_element_type=jnp.float32) + b2_ref[0, 0]
    v1_ref[0] = (ALPHA * jnp.tanh(ALPHA * h1)).astype(jnp.bfloat16)
    v2_ref[0] = jnp.tanh(ALPHA * h2).astype(jnp.bfloat16)


def _t_block(v1s, v2full):
    t1 = lax.dot_general(v1s, v2full, (((1,), (1,)), ((), ())),
                         preferred_element_type=jnp.float32)
    t2 = lax.dot_general(v2full, v1s, (((1,), (1,)), ((), ())),
                         preferred_element_type=jnp.float32)
    t1_16 = t1.astype(jnp.bfloat16)
    t2_16 = t2.astype(jnp.bfloat16)
    return jnp.tanh(t1_16 - t2_16.T)


def _count_gt(c16, mid16):
    c = jnp.where(c16 > mid16, jnp.bfloat16(1.0), jnp.bfloat16(0.0))
    w = c.shape[1]
    while w > 128:
        w //= 2
        c = c[:, :w] + c[:, w:2 * w]
    return jnp.sum(c.astype(jnp.float32), axis=1, keepdims=True)


def _strip_kernel(v1_ref, v2_ref, o_ref):
    t16 = jnp.concatenate(
        [_t_block(v1_ref[0], v2_ref[0]), _t_block(v1_ref[1], v2_ref[1])],
        axis=1)

    lo = jnp.full((STRIP, 1), -(2.0 ** -9), jnp.float32)
    hi = jnp.full((STRIP, 1), 1.0, jnp.float32)
    for _ in range(N_ITERS):
        mid = 0.5 * (lo + hi)
        cnt = _count_gt(t16, mid.astype(jnp.bfloat16))
        ge_k = cnt >= K
        lo = jnp.where(ge_k, mid, lo)
        hi = jnp.where(ge_k, hi, mid)
    thr16 = jnp.maximum(lo, 0.0).astype(jnp.bfloat16)
    o_ref[...] = jnp.where(t16 > thr16, t16,
                           jnp.bfloat16(0.0)).astype(jnp.float32)


def kernel(idx, emb1, emb2, W1, b1, W2, b2):
    nm, n_sub, dim = emb1.shape
    n_mod = 2
    N = n_mod * n_sub

    v1, v2 = pl.pallas_call(
        _dense_tanh_kernel,
        out_shape=(jax.ShapeDtypeStruct((nm, n_sub, dim), jnp.bfloat16),
                   jax.ShapeDtypeStruct((nm, n_sub, dim), jnp.bfloat16)),
        grid=(nm,),
        in_specs=[
            pl.BlockSpec((1, n_sub, dim), lambda b: (b, 0, 0)),
            pl.BlockSpec((1, dim, dim), lambda b: (b, 0, 0)),
            pl.BlockSpec((1, 1, dim), lambda b: (b, 0, 0)),
            pl.BlockSpec((1, n_sub, dim), lambda b: (b, 0, 0)),
            pl.BlockSpec((1, dim, dim), lambda b: (b, 0, 0)),
            pl.BlockSpec((1, 1, dim), lambda b: (b, 0, 0)),
        ],
        out_specs=(pl.BlockSpec((1, n_sub, dim), lambda b: (b, 0, 0)),
                   pl.BlockSpec((1, n_sub, dim), lambda b: (b, 0, 0))),
        compiler_params=pltpu.CompilerParams(
            dimension_semantics=("parallel",)),
    )(emb1, W1, b1.reshape(nm, 1, dim), emb2, W2, b2.reshape(nm, 1, dim))

    strips_per_mod = n_sub // STRIP
    out = pl.pallas_call(
        _strip_kernel,
        out_shape=jax.ShapeDtypeStruct((N, N), jnp.float32),
        grid=(n_mod * strips_per_mod,),
        in_specs=[
            pl.BlockSpec(
                (n_mod, STRIP, dim),
                lambda s: (s // strips_per_mod, s % strips_per_mod, 0)),
            pl.BlockSpec((n_mod, n_sub, dim),
                         lambda s: (s // strips_per_mod, 0, 0)),
        ],
        out_specs=pl.BlockSpec((STRIP, N), lambda s: (s, 0)),
        compiler_params=pltpu.CompilerParams(
            dimension_semantics=("parallel",)),
    )(v1, v2)
    return out


# speedup vs baseline: 80.4530x; 1.0562x over previous
import jax
import jax.numpy as jnp
from jax import lax
from jax.experimental import pallas as pl
from jax.experimental.pallas import tpu as pltpu

ALPHA = 3.0
K = 64
STRIP = 512
N_ITERS = 8


def _dense(e, w, b):
    h = lax.dot_general(e, w, (((1,), (1,)), ((), ())),
                        preferred_element_type=jnp.float32) + b
    return jnp.tanh(ALPHA * h)


def _t_block(v1s, v2full):
    t1 = lax.dot_general(v1s, v2full, (((1,), (1,)), ((), ())),
                         preferred_element_type=jnp.float32)
    t2 = lax.dot_general(v2full, v1s, (((1,), (1,)), ((), ())),
                         preferred_element_type=jnp.float32)
    t1_16 = t1.astype(jnp.bfloat16)
    t2_16 = t2.astype(jnp.bfloat16)
    return jnp.tanh(t1_16 - t2_16.T)


def _count_gt(c16, mid16):
    c = jnp.where(c16 > mid16, jnp.bfloat16(1.0), jnp.bfloat16(0.0))
    w = c.shape[1]
    while w > 128:
        w //= 2
        c = c[:, :w] + c[:, w:2 * w]
    return jnp.sum(c.astype(jnp.float32), axis=1, keepdims=True)


def _fused_kernel(e1_ref, w1_ref, b1_ref, e2_ref, w2_ref, b2_ref,
                  o_ref, v1_s, v2_s):
    strips_per_mod = e1_ref.shape[1] // STRIP
    s = pl.program_id(0)

    @pl.when(s == 0)
    def _():
        for b in range(e1_ref.shape[0]):
            v1_s[b] = (ALPHA * _dense(e1_ref[b], w1_ref[b], b1_ref[b, 0])
                       ).astype(jnp.bfloat16)
            v2_s[b] = _dense(e2_ref[b], w2_ref[b], b2_ref[b, 0]
                             ).astype(jnp.bfloat16)

    i = s // strips_per_mod
    r = s % strips_per_mod
    rows = pl.ds(r * STRIP, STRIP)
    t16 = jnp.concatenate(
        [_t_block(v1_s[2 * i, rows, :], v2_s[2 * i]),
         _t_block(v1_s[2 * i + 1, rows, :], v2_s[2 * i + 1])],
        axis=1)

    lo = jnp.full((STRIP, 1), -(2.0 ** -9), jnp.float32)
    hi = jnp.full((STRIP, 1), 1.0, jnp.float32)
    for _ in range(N_ITERS):
        mid = 0.5 * (lo + hi)
        cnt = _count_gt(t16, mid.astype(jnp.bfloat16))
        ge_k = cnt >= K
        lo = jnp.where(ge_k, mid, lo)
        hi = jnp.where(ge_k, hi, mid)
    thr16 = jnp.maximum(lo, 0.0).astype(jnp.bfloat16)
    o_ref[...] = jnp.where(t16 > thr16, t16,
                           jnp.bfloat16(0.0)).astype(jnp.float32)


def kernel(idx, emb1, emb2, W1, b1, W2, b2):
    nm, n_sub, dim = emb1.shape
    n_mod = 2
    N = n_mod * n_sub
    whole = lambda s: (0, 0, 0)

    out = pl.pallas_call(
        _fused_kernel,
        out_shape=jax.ShapeDtypeStruct((N, N), jnp.float32),
        grid=(n_mod * (n_sub // STRIP),),
        in_specs=[
            pl.BlockSpec((nm, n_sub, dim), whole),
            pl.BlockSpec((nm, dim, dim), whole),
            pl.BlockSpec((nm, 1, dim), whole),
            pl.BlockSpec((nm, n_sub, dim), whole),
            pl.BlockSpec((nm, dim, dim), whole),
            pl.BlockSpec((nm, 1, dim), whole),
        ],
        out_specs=pl.BlockSpec((STRIP, N), lambda s: (s, 0)),
        scratch_shapes=[
            pltpu.VMEM((nm, n_sub, dim), jnp.bfloat16),
            pltpu.VMEM((nm, n_sub, dim), jnp.bfloat16),
        ],
    )(emb1, W1, b1.reshape(nm, 1, dim), emb2, W2, b2.reshape(nm, 1, dim))
    return out
